# baseline (speedup 1.0000x reference)
import jax
import jax.numpy as jnp
from jax import lax
from jax.experimental import pallas as pl
from jax.experimental.pallas import tpu as pltpu
from jax.experimental.pallas import tpu_sc as plsc

_B = 16384
_D = 64
_NC = 2
_NS = 16
_NW = _NC * _NS
_BPW = _B // _NW
_LANES = 16
_VOCAB = 1000000
_TCOL = 128
_NFULL = _VOCAB // _TCOL
_TAIL0 = _NFULL * _TCOL
_DEPTH = 12
_CPW = 252
_FW = 1
_FROWS = _FW * _TCOL
_CPWF = _CPW // _FW
_NFETCH = _NFULL // _FW
_NIT = _CPWF // _DEPTH
_BATCH = 256
_WIN = _BATCH + _LANES
_KPAD = _B + _BATCH + 8
_SLACK = _B * _D
_BIG = 2 ** 29


def _scal(vec_ref, i, iota):
    base = pl.multiple_of(jnp.bitwise_and(i, ~15), 8)
    win = vec_ref[pl.ds(base, _LANES)]
    lane = jnp.bitwise_and(i, 15)
    return lax.reduce_max(jnp.where(iota == lane, win, 0), axes=(0,))


def _extract_run(buf, colmask, chunk_end, kend, carry_in, sorted_hbm,
                 order_hbm, sorted_v, order_v, stage_v, out_hbm, sem, iota):
    def body(carry):
        k, kb, cnt, rr = carry

        def refill():
            kb_new = pl.multiple_of(jnp.bitwise_and(k, ~7), 8)
            pltpu.sync_copy(sorted_hbm.at[pl.ds(kb_new, _BATCH)],
                            sorted_v.at[pl.ds(0, _BATCH)])
            pltpu.sync_copy(order_hbm.at[pl.ds(kb_new, _BATCH)],
                            order_v.at[pl.ds(0, _BATCH)])

        need = (k - kb) >= (_BATCH - 16)
        pl.when(need)(refill)
        kb2 = jnp.where(need, jnp.bitwise_and(k, ~7), kb)

        def drain():
            pltpu.make_async_copy(
                out_hbm.at[pl.ds(0, _BATCH * _D)],
                stage_v.at[pl.ds(0, _BATCH * _D)], sem).wait()

        pl.when(jnp.logical_and(jnp.bitwise_and(cnt, _BATCH - 1) == 0,
                                cnt > 0))(drain)

        idx = k - kb2
        pp = _scal(order_v, idx, iota)
        colv = jnp.broadcast_to(jnp.bitwise_and(rr, colmask), (_LANES,))
        slot = pl.multiple_of(jnp.bitwise_and(cnt, _BATCH - 1) * _D, 8)
        for t in range(_D // _LANES):
            v = plsc.load_gather(buf, [t * _LANES + iota, colv])
            stage_v[pl.ds(pl.multiple_of(slot + t * _LANES, 8), _LANES)] = v
        pltpu.async_copy(stage_v.at[pl.ds(slot, _D)],
                         out_hbm.at[pl.ds(pl.multiple_of(pp * _D, 8), _D)],
                         sem)
        rr_next = _scal(sorted_v, idx + 1, iota)
        return k + 1, kb2, cnt + 1, rr_next

    def cond(carry):
        k, kb, cnt, rr = carry
        return jnp.logical_and(k < kend, rr < chunk_end)

    return lax.while_loop(cond, body, carry_in)


def _stream_table(wid, c0, tab_hbm, tail_hbm, sorted_hbm, order_hbm,
                  kb_hbm, out_hbm, kb_v, sorted_v, order_v,
                  bufs, tbuf, stage_v, bsems, sem, cnt_in, iota):
    pltpu.sync_copy(kb_hbm, kb_v)
    kstart = _scal(kb_v, wid, iota)
    kend = _scal(kb_v, wid + 1, iota)

    def fire(i_local, slot):
        jn = jnp.minimum(c0 + i_local, _NFETCH - 1)
        pltpu.async_copy(
            tab_hbm.at[:, pl.ds(pl.multiple_of(jn * _FROWS, _TCOL), _FROWS)],
            bufs[slot], bsems[slot])

    for s in range(_DEPTH):
        fire(s, s)

    kb0 = pl.multiple_of(jnp.bitwise_and(kstart, ~7), 8)
    pltpu.sync_copy(sorted_hbm.at[pl.ds(kb0, _BATCH)],
                    sorted_v.at[pl.ds(0, _BATCH)])
    pltpu.sync_copy(order_hbm.at[pl.ds(kb0, _BATCH)],
                    order_v.at[pl.ds(0, _BATCH)])
    rr0 = _scal(sorted_v, kstart - kb0, iota)

    def it_body(ii, carry):
        k, kb, cnt, rr = carry
        for s in range(_DEPTH):
            i_local = ii * _DEPTH + s
            pltpu.make_async_copy(
                tab_hbm.at[:, pl.ds(0, _FROWS)], bufs[s], bsems[s]).wait()
            chunk_end = (c0 + i_local + 1) * _FROWS
            k, kb, cnt, rr = _extract_run(
                bufs[s], _FROWS - 1, chunk_end, kend, (k, kb, cnt, rr),
                sorted_hbm, order_hbm, sorted_v, order_v, stage_v, out_hbm,
                sem, iota)
            fire(i_local + _DEPTH, s)
        return k, kb, cnt, rr

    k, kb, cnt, rr = lax.fori_loop(
        0, _NIT, it_body, (kstart, kb0, cnt_in, rr0))
    for s in range(_DEPTH):
        pltpu.make_async_copy(
            tab_hbm.at[:, pl.ds(0, _FROWS)], bufs[s], bsems[s]).wait()

    pltpu.sync_copy(tail_hbm, tbuf)
    kend_t = jnp.where(wid == _NW - 1, jnp.int32(_B), jnp.int32(0))
    k, kb, cnt, rr = _extract_run(
        tbuf, _TCOL - 1, _BIG, kend_t, (k, kb, cnt, rr), sorted_hbm,
        order_hbm, sorted_v, order_v, stage_v, out_hbm, sem, iota)
    return cnt


def _phase1_body(csort_hbm, corder_hbm, ckb_hbm,
                 wsort_hbm, worder_hbm, wkb_hbm,
                 ctab_hbm, wtab_hbm, ctail_hbm, wtail_hbm,
                 cg_hbm, wg_hbm,
                 kb_v, sorted_v, order_v,
                 b0, b1, b2, b3, b4, b5, b6, b7, b8, b9, b10, b11,
                 tbuf, stage_v,
                 s0, s1, s2, s3, s4, s5, s6, s7, s8, s9, s10, s11, ssem):
    wid = lax.axis_index("s") * _NC + lax.axis_index("c")
    c0 = wid * _CPWF
    iota = lax.iota(jnp.int32, _LANES)
    bufs = (b0, b1, b2, b3, b4, b5, b6, b7, b8, b9, b10, b11)
    bsems = (s0, s1, s2, s3, s4, s5, s6, s7, s8, s9, s10, s11)

    cnt = _stream_table(wid, c0, ctab_hbm, ctail_hbm, csort_hbm, corder_hbm,
                        ckb_hbm, cg_hbm, kb_v, sorted_v, order_v,
                        bufs, tbuf, stage_v, bsems, ssem, jnp.int32(0), iota)
    cnt = _stream_table(wid, c0, wtab_hbm, wtail_hbm, wsort_hbm, worder_hbm,
                        wkb_hbm, wg_hbm, kb_v, sorted_v, order_v,
                        bufs, tbuf, stage_v, bsems, ssem, cnt, iota)

    def pad_body(carry):
        cnt2 = carry
        slot = pl.multiple_of(jnp.bitwise_and(cnt2, _BATCH - 1) * _D, 8)
        pltpu.async_copy(stage_v.at[pl.ds(slot, _D)],
                         cg_hbm.at[pl.ds(_SLACK, _D)], ssem)
        return cnt2 + 1

    cnt = lax.while_loop(
        lambda c: jnp.bitwise_and(c, _BATCH - 1) != 0, pad_body, cnt)

    def final_drain():
        pltpu.make_async_copy(
            cg_hbm.at[pl.ds(0, _BATCH * _D)],
            stage_v.at[pl.ds(0, _BATCH * _D)], ssem).wait()

    pl.when(cnt > 0)(final_drain)


def _phase2_body(cg_hbm, wg_hbm, out_hbm, crows_v, wrows_v, res_v):
    wid = lax.axis_index("s") * _NC + lax.axis_index("c")
    base = wid * _BPW
    lane = lax.iota(jnp.int32, _LANES)

    pltpu.sync_copy(cg_hbm.at[pl.ds(base * _D, _BPW * _D)], crows_v)
    pltpu.sync_copy(wg_hbm.at[pl.ds(base * _D, _BPW * _D)], wrows_v)

    def g_body(g, carry):
        flat = ((g * _LANES + lane) * _D).astype(jnp.int32)
        acc = jnp.zeros((_LANES,), jnp.float32)
        for d in range(_D):
            a = plsc.load_gather(crows_v, [flat + d])
            b = plsc.load_gather(wrows_v, [flat + d])
            acc = acc + a * b
        res_v[pl.ds(g * _LANES, _LANES)] = 1.0 / (1.0 + jnp.exp(-acc))
        return carry

    lax.fori_loop(0, _BPW // _LANES, g_body, 0)
    pltpu.sync_copy(res_v, out_hbm.at[pl.ds(base, _BPW)])


def _prep(idx):
    iota_b = jnp.arange(_B, dtype=jnp.int32)
    srt, order = lax.sort_key_val(idx, iota_b)
    edges = jnp.minimum(jnp.arange(48, dtype=jnp.int32) * _CPW,
                        _NFULL) * _TCOL
    kbounds = jnp.sum(srt[None, :] < edges[:, None], axis=1,
                      dtype=jnp.int32)
    srt_p = jnp.pad(srt, (0, _KPAD - _B), constant_values=2 ** 30)
    ord_p = jnp.pad(order, (0, _KPAD - _B))
    return srt_p, ord_p, kbounds


@jax.jit
def kernel(c, w, c_table, w_table):
    mesh = plsc.VectorSubcoreMesh(core_axis_name="c", subcore_axis_name="s")
    p1 = pl.kernel(
        _phase1_body,
        out_type=(jax.ShapeDtypeStruct((_SLACK + _D,), jnp.float32),
                  jax.ShapeDtypeStruct((_SLACK + _D,), jnp.float32)),
        mesh=mesh,
        scratch_types=(
            [pltpu.VMEM((48,), jnp.int32),
             pltpu.VMEM((_WIN,), jnp.int32),
             pltpu.VMEM((_WIN,), jnp.int32)]
            + [pltpu.VMEM((_D, _FROWS), jnp.float32)] * _DEPTH
            + [pltpu.VMEM((_D, _TCOL), jnp.float32)]
            + [pltpu.VMEM((_BATCH * _D,), jnp.float32)]
            + [pltpu.SemaphoreType.DMA] * (_DEPTH + 1)
        ),
        compiler_params=pltpu.CompilerParams(
            needs_layout_passes=False, use_tc_tiling_on_sc=True),
    )
    p2 = pl.kernel(
        _phase2_body,
        out_type=jax.ShapeDtypeStruct((_B,), jnp.float32),
        mesh=mesh,
        scratch_types=[
            pltpu.VMEM((_BPW * _D,), jnp.float32),
            pltpu.VMEM((_BPW * _D,), jnp.float32),
            pltpu.VMEM((_BPW,), jnp.float32),
        ],
        compiler_params=pltpu.CompilerParams(
            needs_layout_passes=False, use_tc_tiling_on_sc=False),
    )

    c = c.astype(jnp.int32)
    w = w.astype(jnp.int32)
    cs, co, ckb = _prep(c)
    ws, wo, wkb = _prep(w)
    ctail = jnp.pad(c_table[_TAIL0:].T, ((0, 0), (0, _TCOL - _D)))
    wtail = jnp.pad(w_table[_TAIL0:].T, ((0, 0), (0, _TCOL - _D)))
    cg, wg = p1(cs, co, ckb, ws, wo, wkb, c_table.T, w_table.T,
                ctail, wtail)
    return p2(cg, wg)


# speedup vs baseline: 1.1240x; 1.1240x over previous
import jax
import jax.numpy as jnp
from jax import lax
from jax.experimental import pallas as pl
from jax.experimental.pallas import tpu as pltpu
from jax.experimental.pallas import tpu_sc as plsc

_B = 16384
_D = 64
_NC = 2
_NS = 16
_NW = _NC * _NS
_BPW = _B // _NW
_LANES = 16
_VOCAB = 1000000
_TCOL = 128
_NFULL = _VOCAB // _TCOL
_TAIL0 = _NFULL * _TCOL
_DEPTH = 8
_CPW = 248
_FW = 1
_FROWS = _FW * _TCOL
_CPWF = _CPW // _FW
_NFETCH = _NFULL // _FW
_NIT = _CPWF // _DEPTH
_BATCH = 256
_WIN = _BATCH + _LANES
_KPAD = _B + _BATCH + 8
_SLACK = _B * _D
_BIG = 2 ** 29


def _scal(vec_ref, i, iota):
    base = pl.multiple_of(jnp.bitwise_and(i, ~15), 8)
    win = vec_ref[pl.ds(base, _LANES)]
    lane = jnp.bitwise_and(i, 15)
    return lax.reduce_max(jnp.where(iota == lane, win, 0), axes=(0,))


def _extract_run(buf, colmask, chunk_end, kend, carry_in, sorted_hbm,
                 order_hbm, sorted_v, order_v, stage_v, out_hbm, sem, iota):
    def body(carry):
        k, kb, cnt, rr = carry

        def refill():
            kb_new = pl.multiple_of(jnp.bitwise_and(k, ~7), 8)
            pltpu.sync_copy(sorted_hbm.at[pl.ds(kb_new, _BATCH)],
                            sorted_v.at[pl.ds(0, _BATCH)])
            pltpu.sync_copy(order_hbm.at[pl.ds(kb_new, _BATCH)],
                            order_v.at[pl.ds(0, _BATCH)])

        need = (k - kb) >= (_BATCH - 16)
        pl.when(need)(refill)
        kb2 = jnp.where(need, jnp.bitwise_and(k, ~7), kb)

        def drain():
            pltpu.make_async_copy(
                out_hbm.at[pl.ds(0, _BATCH * _D)],
                stage_v.at[pl.ds(0, _BATCH * _D)], sem).wait()

        pl.when(jnp.logical_and(jnp.bitwise_and(cnt, _BATCH - 1) == 0,
                                cnt > 0))(drain)

        idx = k - kb2
        pp = _scal(order_v, idx, iota)
        colv = jnp.broadcast_to(jnp.bitwise_and(rr, colmask), (_LANES,))
        slot = pl.multiple_of(jnp.bitwise_and(cnt, _BATCH - 1) * _D, 8)
        for t in range(_D // _LANES):
            v = plsc.load_gather(buf, [t * _LANES + iota, colv])
            stage_v[pl.ds(pl.multiple_of(slot + t * _LANES, 8), _LANES)] = v
        pltpu.async_copy(stage_v.at[pl.ds(slot, _D)],
                         out_hbm.at[pl.ds(pl.multiple_of(pp * _D, 8), _D)],
                         sem)
        rr_next = _scal(sorted_v, idx + 1, iota)
        return k + 1, kb2, cnt + 1, rr_next

    def cond(carry):
        k, kb, cnt, rr = carry
        return jnp.logical_and(k < kend, rr < chunk_end)

    return lax.while_loop(cond, body, carry_in)


def _stream_table(wid, c0, tab_hbm, tail_hbm, sorted_hbm, order_hbm,
                  kb_hbm, out_hbm, kb_v, sorted_v, order_v,
                  bufs, tbuf, stage_v, bsems, sem, cnt_in, iota):
    pltpu.sync_copy(kb_hbm, kb_v)
    kstart = _scal(kb_v, wid, iota)
    kend = _scal(kb_v, wid + 1, iota)

    def fire(i_local, slot):
        jn = jnp.minimum(c0 + i_local, _NFETCH - 1)
        pltpu.async_copy(
            tab_hbm.at[:, pl.ds(pl.multiple_of(jn * _FROWS, _TCOL), _FROWS)],
            bufs[slot], bsems[slot])

    for s in range(_DEPTH):
        fire(s, s)

    kb0 = pl.multiple_of(jnp.bitwise_and(kstart, ~7), 8)
    pltpu.sync_copy(sorted_hbm.at[pl.ds(kb0, _BATCH)],
                    sorted_v.at[pl.ds(0, _BATCH)])
    pltpu.sync_copy(order_hbm.at[pl.ds(kb0, _BATCH)],
                    order_v.at[pl.ds(0, _BATCH)])
    rr0 = _scal(sorted_v, kstart - kb0, iota)

    def it_body(ii, carry):
        k, kb, cnt, rr = carry
        for s in range(_DEPTH):
            i_local = ii * _DEPTH + s
            pltpu.make_async_copy(
                tab_hbm.at[:, pl.ds(0, _FROWS)], bufs[s], bsems[s]).wait()
            chunk_end = (c0 + i_local + 1) * _FROWS
            k, kb, cnt, rr = _extract_run(
                bufs[s], _FROWS - 1, chunk_end, kend, (k, kb, cnt, rr),
                sorted_hbm, order_hbm, sorted_v, order_v, stage_v, out_hbm,
                sem, iota)
            fire(i_local + _DEPTH, s)
        return k, kb, cnt, rr

    k, kb, cnt, rr = lax.fori_loop(
        0, _NIT, it_body, (kstart, kb0, cnt_in, rr0))
    for s in range(_DEPTH):
        pltpu.make_async_copy(
            tab_hbm.at[:, pl.ds(0, _FROWS)], bufs[s], bsems[s]).wait()

    pltpu.sync_copy(tail_hbm, tbuf)
    kend_t = jnp.where(wid == _NW - 1, jnp.int32(_B), jnp.int32(0))
    k, kb, cnt, rr = _extract_run(
        tbuf, _TCOL - 1, _BIG, kend_t, (k, kb, cnt, rr), sorted_hbm,
        order_hbm, sorted_v, order_v, stage_v, out_hbm, sem, iota)
    return cnt


def _phase1_body(csort_hbm, corder_hbm, ckb_hbm,
                 wsort_hbm, worder_hbm, wkb_hbm,
                 ctab_hbm, wtab_hbm, ctail_hbm, wtail_hbm,
                 cg_hbm, wg_hbm,
                 kb_v, sorted_v, order_v,
                 b0, b1, b2, b3, b4, b5, b6, b7, tbuf, stage_v,
                 s0, s1, s2, s3, s4, s5, s6, s7, ssem):
    wid = lax.axis_index("s") * _NC + lax.axis_index("c")
    c0 = wid * _CPWF
    iota = lax.iota(jnp.int32, _LANES)
    bufs = (b0, b1, b2, b3, b4, b5, b6, b7)
    bsems = (s0, s1, s2, s3, s4, s5, s6, s7)

    cnt = _stream_table(wid, c0, ctab_hbm, ctail_hbm, csort_hbm, corder_hbm,
                        ckb_hbm, cg_hbm, kb_v, sorted_v, order_v,
                        bufs, tbuf, stage_v, bsems, ssem, jnp.int32(0), iota)
    cnt = _stream_table(wid, c0, wtab_hbm, wtail_hbm, wsort_hbm, worder_hbm,
                        wkb_hbm, wg_hbm, kb_v, sorted_v, order_v,
                        bufs, tbuf, stage_v, bsems, ssem, cnt, iota)

    def pad_body(carry):
        cnt2 = carry
        slot = pl.multiple_of(jnp.bitwise_and(cnt2, _BATCH - 1) * _D, 8)
        pltpu.async_copy(stage_v.at[pl.ds(slot, _D)],
                         cg_hbm.at[pl.ds(_SLACK, _D)], ssem)
        return cnt2 + 1

    cnt = lax.while_loop(
        lambda c: jnp.bitwise_and(c, _BATCH - 1) != 0, pad_body, cnt)

    def final_drain():
        pltpu.make_async_copy(
            cg_hbm.at[pl.ds(0, _BATCH * _D)],
            stage_v.at[pl.ds(0, _BATCH * _D)], ssem).wait()

    pl.when(cnt > 0)(final_drain)


def _phase2_body(cg_hbm, wg_hbm, out_hbm, crows_v, wrows_v, res_v):
    wid = lax.axis_index("s") * _NC + lax.axis_index("c")
    base = wid * _BPW
    lane = lax.iota(jnp.int32, _LANES)

    pltpu.sync_copy(cg_hbm.at[pl.ds(base * _D, _BPW * _D)], crows_v)
    pltpu.sync_copy(wg_hbm.at[pl.ds(base * _D, _BPW * _D)], wrows_v)

    def g_body(g, carry):
        flat = ((g * _LANES + lane) * _D).astype(jnp.int32)
        acc = jnp.zeros((_LANES,), jnp.float32)
        for d in range(_D):
            a = plsc.load_gather(crows_v, [flat + d])
            b = plsc.load_gather(wrows_v, [flat + d])
            acc = acc + a * b
        res_v[pl.ds(g * _LANES, _LANES)] = 1.0 / (1.0 + jnp.exp(-acc))
        return carry

    lax.fori_loop(0, _BPW // _LANES, g_body, 0)
    pltpu.sync_copy(res_v, out_hbm.at[pl.ds(base, _BPW)])


def _prep(idx):
    iota_b = jnp.arange(_B, dtype=jnp.int32)
    srt, order = lax.sort_key_val(idx, iota_b)
    edges = jnp.minimum(jnp.arange(48, dtype=jnp.int32) * _CPW,
                        _NFULL) * _TCOL
    kbounds = jnp.sum(srt[None, :] < edges[:, None], axis=1,
                      dtype=jnp.int32)
    srt_p = jnp.pad(srt, (0, _KPAD - _B), constant_values=2 ** 30)
    ord_p = jnp.pad(order, (0, _KPAD - _B))
    return srt_p, ord_p, kbounds


@jax.jit
def kernel(c, w, c_table, w_table):
    mesh = plsc.VectorSubcoreMesh(core_axis_name="c", subcore_axis_name="s")
    p1 = pl.kernel(
        _phase1_body,
        out_type=(jax.ShapeDtypeStruct((_SLACK + _D,), jnp.float32),
                  jax.ShapeDtypeStruct((_SLACK + _D,), jnp.float32)),
        mesh=mesh,
        scratch_types=(
            [pltpu.VMEM((48,), jnp.int32),
             pltpu.VMEM((_WIN,), jnp.int32),
             pltpu.VMEM((_WIN,), jnp.int32)]
            + [pltpu.VMEM((_D, _FROWS), jnp.float32)] * _DEPTH
            + [pltpu.VMEM((_D, _TCOL), jnp.float32)]
            + [pltpu.VMEM((_BATCH * _D,), jnp.float32)]
            + [pltpu.SemaphoreType.DMA] * (_DEPTH + 1)
        ),
        compiler_params=pltpu.CompilerParams(
            needs_layout_passes=False, use_tc_tiling_on_sc=True),
    )
    p2 = pl.kernel(
        _phase2_body,
        out_type=jax.ShapeDtypeStruct((_B,), jnp.float32),
        mesh=mesh,
        scratch_types=[
            pltpu.VMEM((_BPW * _D,), jnp.float32),
            pltpu.VMEM((_BPW * _D,), jnp.float32),
            pltpu.VMEM((_BPW,), jnp.float32),
        ],
        compiler_params=pltpu.CompilerParams(
            needs_layout_passes=False, use_tc_tiling_on_sc=False),
    )

    c = c.astype(jnp.int32)
    w = w.astype(jnp.int32)
    cs, co, ckb = _prep(c)
    ws, wo, wkb = _prep(w)
    ctail = jnp.pad(c_table[_TAIL0:].T, ((0, 0), (0, _TCOL - _D)))
    wtail = jnp.pad(w_table[_TAIL0:].T, ((0, 0), (0, _TCOL - _D)))
    cg, wg = p1(cs, co, ckb, ws, wo, wkb, c_table.T, w_table.T,
                ctail, wtail)
    return p2(cg, wg)
